# Initial kernel scaffold; baseline (speedup 1.0000x reference)
import functools

import jax
import jax.numpy as jnp
from jax import lax
from jax.experimental import pallas as pl
from jax.experimental.pallas import tpu as pltpu
from jax.experimental.pallas import tpu_sc as plsc

N0 = 10000
E = 160000
D = 128
NP0 = 10240
EP = 163840
N1, NP1 = 5000, 5120
N2, NP2 = 2500, 2560
NEG = jnp.float32(-3.0e38)

_MESH = plsc.VectorSubcoreMesh(core_axis_name="c", subcore_axis_name="s")


def _sc_edge_stats(src_h, dst_h, ones_h, out_h, sv, dv, mv, cnt_sh, self_sh,
                   ones_v):
    c = lax.axis_index("c")
    tid = lax.axis_index("s")
    wid = tid * 2 + c
    DUMP = NP0

    pltpu.sync_copy(ones_h.at[1], ones_v)
    for k in range(6):
        base = tid * 768 + k * 128
        pltpu.sync_copy(ones_v, cnt_sh.at[pl.ds(base, 128)])
        pltpu.sync_copy(ones_v, self_sh.at[pl.ds(base, 128)])
    pltpu.sync_copy(ones_h.at[0], ones_v)
    plsc.subcore_barrier()

    ebase = wid * 5120

    def body(b, carry):
        off = ebase + b * 128
        pltpu.sync_copy(src_h.at[pl.ds(off, 128)], sv)
        pltpu.sync_copy(dst_h.at[pl.ds(off, 128)], dv)
        pltpu.sync_copy(ones_v, cnt_sh.at[dv], add=True)
        for kk in range(8):
            s16 = sv[pl.ds(kk * 16, 16)]
            d16 = dv[pl.ds(kk * 16, 16)]
            sel = jnp.where(s16 == d16, d16, jnp.full((16,), DUMP, jnp.int32))
            mv[pl.ds(kk * 16, 16)] = sel
        pltpu.sync_copy(ones_v, self_sh.at[mv], add=True)
        return carry

    lax.fori_loop(0, 40, body, 0)
    plsc.subcore_barrier()
    lo = tid * 640
    pltpu.sync_copy(cnt_sh.at[pl.ds(lo, 640)], out_h.at[c, 0, pl.ds(lo, 640)])
    pltpu.sync_copy(self_sh.at[pl.ds(lo, 640)], out_h.at[c, 1, pl.ds(lo, 640)])


def _edge_stats(src, dst, ones2):
    k = functools.partial(
        pl.kernel, mesh=_MESH,
        out_type=jax.ShapeDtypeStruct((2, 2, NP0, 16), jnp.float32),
        scratch_types=[
            pltpu.VMEM((128,), jnp.int32),
            pltpu.VMEM((128,), jnp.int32),
            pltpu.VMEM((128,), jnp.int32),
            pltpu.VMEM_SHARED((12288, 16), jnp.float32),
            pltpu.VMEM_SHARED((12288, 16), jnp.float32),
            pltpu.VMEM((128, 16), jnp.float32),
        ],
    )(_sc_edge_stats)
    return k(src, dst, ones2)


def _sc_scatter_feat(src_h, dst_h, zeros_h, hs_h, out_h, sv, dv, rows_v,
                     zrow_v, zacc_sh, sem):
    c = lax.axis_index("c")
    tid = lax.axis_index("s")
    wid = tid * 2 + c

    pltpu.sync_copy(zeros_h, zrow_v)
    for k in range(6):
        base = tid * 768 + k * 128
        pltpu.sync_copy(zrow_v, zacc_sh.at[pl.ds(base, 128)])
    plsc.subcore_barrier()

    ebase = wid * 5120

    def body(b, carry):
        off = ebase + b * 128
        pltpu.sync_copy(src_h.at[pl.ds(off, 128)], sv)
        pltpu.sync_copy(dst_h.at[pl.ds(off, 128)], dv)
        pltpu.async_copy(hs_h.at[sv], rows_v, sem).wait()
        pltpu.sync_copy(rows_v, zacc_sh.at[dv], add=True)
        return carry

    lax.fori_loop(0, 40, body, 0)
    plsc.subcore_barrier()
    lo = tid * 640
    pltpu.sync_copy(zacc_sh.at[pl.ds(lo, 640)], out_h.at[c, pl.ds(lo, 640)])


def _scatter_feat(src, dst, zeros_row, hs):
    k = functools.partial(
        pl.kernel, mesh=_MESH,
        out_type=jax.ShapeDtypeStruct((2, NP0, D), jnp.float32),
        scratch_types=[
            pltpu.VMEM((128,), jnp.int32),
            pltpu.VMEM((128,), jnp.int32),
            pltpu.VMEM((128, D), jnp.float32),
            pltpu.VMEM((128, D), jnp.float32),
            pltpu.VMEM_SHARED((12288, D), jnp.float32),
            pltpu.SemaphoreType.DMA,
        ],
    )(_sc_scatter_feat)
    return k(src, dst, zeros_row, hs)


_LROWS = 102400
_LBUF = 104448
_LDUMP = 102400


def _sc_build_lr(src_h, dst_h, rank_h, perm_h, zeros16_h, lout_h, rout_h,
                 sv, dv, rank_v, perm_v, fkey_v, cidx_v, dbuf_v, z16_v,
                 lr_sh):
    c = lax.axis_index("c")
    tid = lax.axis_index("s")
    iota = lax.iota(jnp.int32, 16)
    ones16 = jnp.full((16,), 1.0, jnp.float32)
    zero16 = jnp.zeros((16,), jnp.float32)
    dump16 = jnp.full((16,), _LDUMP, jnp.int32)
    neg16 = jnp.full((16,), -1, jnp.int32)

    pltpu.sync_copy(rank_h, rank_v)
    pltpu.sync_copy(perm_h, perm_v)
    pltpu.sync_copy(zeros16_h, z16_v)
    ebase = tid * NP0
    pltpu.sync_copy(src_h.at[pl.ds(ebase, NP0)], sv)
    pltpu.sync_copy(dst_h.at[pl.ds(ebase, NP0)], dv)

    def pre_edge(ch, carry):
        s16 = sv[pl.ds(ch * 16, 16)]
        d16 = dv[pl.ds(ch * 16, 16)]
        dc = jnp.minimum(d16, jnp.full((16,), NP0 - 1, jnp.int32))

        @pl.when(c == 0)
        def _():
            a16 = plsc.load_gather(rank_v, [dc])
            ok = (a16 >= 0) & (s16 != d16)
            fkey_v[pl.ds(ch * 16, 16)] = jnp.where(ok, a16 * NP0 + s16, neg16)

        @pl.when(c == 1)
        def _():
            r16 = plsc.load_gather(rank_v, [s16])
            ok = (r16 >= 0) & (s16 != d16)
            fkey_v[pl.ds(ch * 16, 16)] = jnp.where(ok, d16 * NP1 + r16, neg16)

        return carry

    lax.fori_loop(0, NP0 // 16, pre_edge, 0)

    def pre_diag(ch, carry):
        aj = tid * 320 + ch * 16 + iota
        p16 = plsc.load_gather(perm_v, [aj])
        ok = aj < N1

        @pl.when(c == 0)
        def _():
            fkey_v[pl.ds(NP0 + ch * 16, 16)] = jnp.where(
                ok, aj * NP0 + p16, neg16)

        @pl.when(c == 1)
        def _():
            fkey_v[pl.ds(NP0 + ch * 16, 16)] = jnp.where(
                ok, p16 * NP1 + aj, neg16)

        return carry

    lax.fori_loop(0, 20, pre_diag, 0)

    NCH = (NP0 + 320) // 16

    def clean(hi_row):
        def zb(i, carry):
            dbuf_v[i] = zero16
            return carry
        lax.fori_loop(0, hi_row, zb, 0)

    def reset_cidx():
        for k in range(8):
            cidx_v[pl.ds(k * 16, 16)] = dump16

    def stripe(st, carry):
        for k in range(51):
            base = tid * 6528 + k * 128
            pltpu.sync_copy(z16_v, lr_sh.at[pl.ds(base, 128)])
        plsc.subcore_barrier()
        reset_cidx()
        clean(128)
        lo = st * (_LROWS * 16)
        hi = lo + _LROWS * 16

        def chunk(ch, cur):
            k16 = fkey_v[pl.ds(ch * 16, 16)]
            m = (k16 >= lo) & (k16 < hi)
            rel = k16 - lo
            fl = jnp.right_shift(rel, 4)
            ln = jnp.bitwise_and(rel, 15)
            cs = plsc.cumsum(jnp.where(m, jnp.full((16,), 1, jnp.int32),
                                       jnp.zeros((16,), jnp.int32)))
            nv = jnp.max(cs)
            slot = cur + cs - 1
            plsc.store_scatter(cidx_v, [slot], fl, mask=m)
            plsc.store_scatter(dbuf_v, [slot, ln], ones16, mask=m)
            cur2 = cur + nv

            @pl.when(cur2 >= 112)
            def _():
                pltpu.sync_copy(dbuf_v, lr_sh.at[cidx_v], add=True)
                reset_cidx()
                clean(cur2)

            return jnp.where(cur2 >= 112, jnp.int32(0), cur2)

        cur = lax.fori_loop(0, NCH, chunk, jnp.int32(0))

        @pl.when(cur > 0)
        def _():
            pltpu.sync_copy(dbuf_v, lr_sh.at[cidx_v], add=True)
            reset_cidx()
            clean(cur)

        plsc.subcore_barrier()
        tlo = tid * 6400

        @pl.when(c == 0)
        def _():
            pltpu.sync_copy(lr_sh.at[pl.ds(tlo, 6400)],
                            lout_h.at[pl.ds(st * _LROWS + tlo, 6400)])

        @pl.when(c == 1)
        def _():
            pltpu.sync_copy(lr_sh.at[pl.ds(tlo, 6400)],
                            rout_h.at[pl.ds(st * _LROWS + tlo, 6400)])

        plsc.subcore_barrier()
        return carry

    lax.fori_loop(0, 32, stripe, 0)


def _build_lr(src, dst, rank1, perm1, zeros16):
    k = functools.partial(
        pl.kernel, mesh=_MESH,
        out_type=[
            jax.ShapeDtypeStruct((NP1 * NP0 // 16, 16), jnp.float32),
            jax.ShapeDtypeStruct((NP0 * NP1 // 16, 16), jnp.float32),
        ],
        scratch_types=[
            pltpu.VMEM((NP0,), jnp.int32),
            pltpu.VMEM((NP0,), jnp.int32),
            pltpu.VMEM((NP0,), jnp.int32),
            pltpu.VMEM((NP1,), jnp.int32),
            pltpu.VMEM((NP0 + 320,), jnp.int32),
            pltpu.VMEM((128,), jnp.int32),
            pltpu.VMEM((128, 16), jnp.float32),
            pltpu.VMEM((128, 16), jnp.float32),
            pltpu.VMEM_SHARED((_LBUF, 16), jnp.float32),
        ],
    )(_sc_build_lr)
    return k(src, dst, rank1, perm1, zeros16)


def _make_gather(n_rows, n_cols, diag, batch):
    per_w = n_rows // 32
    nb = per_w // batch
    iota = lax.iota(jnp.int32, 16)

    def body(table_h, idx_h, out_h, idx_v, rows_v, sem):
        c = lax.axis_index("c")
        tid = lax.axis_index("s")
        wid = tid * 2 + c
        base = wid * per_w
        pltpu.sync_copy(idx_h.at[pl.ds(base, per_w)], idx_v)

        def b_loop(b, carry):
            pltpu.async_copy(table_h.at[idx_v.at[pl.ds(b * batch, batch)]],
                             rows_v, sem).wait()
            if diag:
                for kk in range(batch // 16):
                    rows16 = kk * 16 + iota
                    cols16 = idx_v[pl.ds(b * batch + kk * 16, 16)]
                    plsc.addupdate_scatter(
                        rows_v, [rows16, cols16],
                        jnp.full((16,), 1.0, jnp.float32))
            pltpu.sync_copy(rows_v, out_h.at[pl.ds(base + b * batch, batch)])
            return carry

        lax.fori_loop(0, nb, b_loop, 0)

    k = functools.partial(
        pl.kernel, mesh=_MESH,
        out_type=jax.ShapeDtypeStruct((n_rows, n_cols), jnp.float32),
        scratch_types=[
            pltpu.VMEM((per_w,), jnp.int32),
            pltpu.VMEM((batch, n_cols), jnp.float32),
            pltpu.SemaphoreType.DMA,
        ],
    )(body)
    return k


def _tc_linear(xin, W, dinv, vals=None):
    n = xin.shape[0]

    def body(x_ref, w_ref, d_ref, v_ref, g_ref, gs_ref):
        xb = x_ref[...]
        if vals is not None:
            xb = xb * v_ref[...][:, None]
        g = jnp.dot(xb, w_ref[...], preferred_element_type=jnp.float32)
        g_ref[...] = g
        gs_ref[...] = g * d_ref[...][:, None]

    vv = vals if vals is not None else dinv
    return pl.pallas_call(
        body,
        grid=(n // 512,),
        in_specs=[
            pl.BlockSpec((512, D), lambda i: (i, 0)),
            pl.BlockSpec((D, D), lambda i: (0, 0)),
            pl.BlockSpec((512,), lambda i: (i,)),
            pl.BlockSpec((512,), lambda i: (i,)),
        ],
        out_specs=[
            pl.BlockSpec((512, D), lambda i: (i, 0)),
            pl.BlockSpec((512, D), lambda i: (i, 0)),
        ],
        out_shape=[
            jax.ShapeDtypeStruct((n, D), jnp.float32),
            jax.ShapeDtypeStruct((n, D), jnp.float32),
        ],
    )(xin, W, dinv, vv)


def _tc_gcn_apply(A, gs, dinv, b, relu, score_w=None):
    n = A.shape[0]
    kb = n // 512

    def body(a_ref, gs_ref, gsb_ref, d_ref, b_ref, w_ref, o_ref, y_ref):
        k = pl.program_id(1)

        @pl.when(k == 0)
        def _():
            o_ref[...] = jnp.zeros_like(o_ref)

        o_ref[...] += jnp.dot(a_ref[...], gs_ref[...],
                              preferred_element_type=jnp.float32)

        @pl.when(k == kb - 1)
        def _():
            out = d_ref[...][:, None] * (o_ref[...] + gsb_ref[...]) \
                + b_ref[...]
            if relu:
                out = jnp.maximum(out, 0.0)
            o_ref[...] = out
            if score_w is not None:
                w = w_ref[...]
                nrm = jnp.sqrt(jnp.sum(w * w))
                y_ref[...] = jnp.tanh(
                    jnp.dot(out, (w / nrm).reshape(D, 1),
                            preferred_element_type=jnp.float32))

    w_in = score_w if score_w is not None else jnp.zeros((1, D), jnp.float32)
    out, y = pl.pallas_call(
        body,
        grid=(n // 512, kb),
        in_specs=[
            pl.BlockSpec((512, 512), lambda i, k: (i, k)),
            pl.BlockSpec((512, D), lambda i, k: (k, 0)),
            pl.BlockSpec((512, D), lambda i, k: (i, 0)),
            pl.BlockSpec((512,), lambda i, k: (i,)),
            pl.BlockSpec((1, D), lambda i, k: (0, 0)),
            pl.BlockSpec((1, D), lambda i, k: (0, 0)),
        ],
        out_specs=[
            pl.BlockSpec((512, D), lambda i, k: (i, 0)),
            pl.BlockSpec((512, 1), lambda i, k: (i, 0)),
        ],
        out_shape=[
            jax.ShapeDtypeStruct((n, D), jnp.float32),
            jax.ShapeDtypeStruct((n, 1), jnp.float32),
        ],
    )(A, gs, gs, dinv, b.reshape(1, D), w_in)
    return out, y[:, 0]


def _tc_rank(y, n_valid, k_keep):
    n = y.shape[0]
    ch = 1024
    nch = n // ch

    def body(yi_ref, yf_ref, r_ref, rm_ref):
        i = pl.program_id(0)
        ig = i * 512 + lax.broadcasted_iota(jnp.int32, (512,), 0)
        yiv = jnp.where(ig < n_valid, yi_ref[...], NEG)

        def cbody(cc, acc):
            yj = yf_ref[pl.ds(cc * ch, ch)]
            jg = cc * ch + lax.broadcasted_iota(jnp.int32, (ch,), 0)
            yjv = jnp.where(jg < n_valid, yj, NEG)
            gt = yjv[None, :] > yiv[:, None]
            tie = (yjv[None, :] == yiv[:, None]) & (jg[None, :] < ig[:, None])
            return acc + jnp.sum((gt | tie).astype(jnp.int32), axis=1)

        r = lax.fori_loop(0, nch, cbody, jnp.zeros((512,), jnp.int32))
        r = jnp.where(ig < n_valid, r, jnp.full((512,), n, jnp.int32))
        r_ref[...] = r
        rm_ref[...] = jnp.where(r < k_keep, r,
                                jnp.full((512,), -1, jnp.int32))

    return pl.pallas_call(
        body,
        grid=(n // 512,),
        in_specs=[
            pl.BlockSpec((512,), lambda i: (i,)),
            pl.BlockSpec((n,), lambda i: (0,)),
        ],
        out_specs=[
            pl.BlockSpec((512,), lambda i: (i,)),
            pl.BlockSpec((512,), lambda i: (i,)),
        ],
        out_shape=[
            jax.ShapeDtypeStruct((n,), jnp.int32),
            jax.ShapeDtypeStruct((n,), jnp.int32),
        ],
    )(y, y)


def _tc_perm_vals(rank, y, np_out):
    n = rank.shape[0]
    ch = 1024
    nch = n // ch

    def body(r_ref, y_ref, p_ref, v_ref):
        j = pl.program_id(0)
        jg = j * 512 + lax.broadcasted_iota(jnp.int32, (512,), 0)

        def cbody(cc, acc):
            pacc, vacc = acc
            rr = r_ref[pl.ds(cc * ch, ch)]
            yy = y_ref[pl.ds(cc * ch, ch)]
            ig = cc * ch + lax.broadcasted_iota(jnp.int32, (ch,), 0)
            eq = (rr[:, None] == jg[None, :]).astype(jnp.float32)
            pacc = pacc + jnp.sum(eq * ig[:, None].astype(jnp.float32),
                                  axis=0)
            vacc = vacc + jnp.sum(eq * yy[:, None], axis=0)
            return (pacc, vacc)

        p, v = lax.fori_loop(0, nch, cbody,
                             (jnp.zeros((512,), jnp.float32),
                              jnp.zeros((512,), jnp.float32)))
        p_ref[...] = p.astype(jnp.int32)
        v_ref[...] = v

    return pl.pallas_call(
        body,
        grid=(np_out // 512,),
        in_specs=[
            pl.BlockSpec((n,), lambda j: (0,)),
            pl.BlockSpec((n,), lambda j: (0,)),
        ],
        out_specs=[
            pl.BlockSpec((512,), lambda j: (j,)),
            pl.BlockSpec((512,), lambda j: (j,)),
        ],
        out_shape=[
            jax.ShapeDtypeStruct((np_out,), jnp.int32),
            jax.ShapeDtypeStruct((np_out,), jnp.float32),
        ],
    )(rank, y)


def _tc_mm_lr(Lf, Rf):
    M, K = NP1, NP0
    BM, BN, BK = 512, 512, 1024
    gk = K // BK

    def body(l_ref, r_ref, o_ref, t_ref):
        k = pl.program_id(2)

        @pl.when(k == 0)
        def _():
            o_ref[...] = jnp.zeros_like(o_ref)

        o_ref[...] += jnp.dot(l_ref[...].astype(jnp.bfloat16),
                              r_ref[...].astype(jnp.bfloat16),
                              preferred_element_type=jnp.float32)

        @pl.when(k == gk - 1)
        def _():
            i = pl.program_id(0)
            j = pl.program_id(1)
            rg = i * BM + lax.broadcasted_iota(jnp.int32, (BM, BN), 0)
            cg = j * BN + lax.broadcasted_iota(jnp.int32, (BM, BN), 1)
            acc = jnp.where(rg == cg, 0.0, o_ref[...])
            o_ref[...] = acc
            t_ref[...] = acc.T

    return pl.pallas_call(
        body,
        grid=(M // BM, M // BN, gk),
        in_specs=[
            pl.BlockSpec((BM, BK), lambda i, j, k: (i, k)),
            pl.BlockSpec((BK, BN), lambda i, j, k: (k, j)),
        ],
        out_specs=[
            pl.BlockSpec((BM, BN), lambda i, j, k: (i, j)),
            pl.BlockSpec((BN, BM), lambda i, j, k: (j, i)),
        ],
        out_shape=[
            jax.ShapeDtypeStruct((M, M), jnp.float32),
            jax.ShapeDtypeStruct((M, M), jnp.float32),
        ],
    )(Lf, Rf)


def _tc_mm_nt(L2, R2T):
    M, K = NP2, NP1
    BM, BN, BK = 512, 512, 1024
    gk = K // BK

    def body(l_ref, r_ref, o_ref):
        k = pl.program_id(2)

        @pl.when(k == 0)
        def _():
            o_ref[...] = jnp.zeros_like(o_ref)

        o_ref[...] += lax.dot_general(
            l_ref[...].astype(jnp.bfloat16), r_ref[...].astype(jnp.bfloat16),
            (((1,), (1,)), ((), ())), preferred_element_type=jnp.float32)

        @pl.when(k == gk - 1)
        def _():
            i = pl.program_id(0)
            j = pl.program_id(1)
            rg = i * BM + lax.broadcasted_iota(jnp.int32, (BM, BN), 0)
            cg = j * BN + lax.broadcasted_iota(jnp.int32, (BM, BN), 1)
            bad = (rg == cg) | (rg >= N2) | (cg >= N2)
            o_ref[...] = jnp.where(bad, 0.0, o_ref[...])

    return pl.pallas_call(
        body,
        grid=(M // BM, M // BN, gk),
        in_specs=[
            pl.BlockSpec((BM, BK), lambda i, j, k: (i, k)),
            pl.BlockSpec((BN, BK), lambda i, j, k: (j, k)),
        ],
        out_specs=pl.BlockSpec((BM, BN), lambda i, j, k: (i, j)),
        out_shape=jax.ShapeDtypeStruct((M, M), jnp.float32),
    )(L2, R2T)


def _tc_deginv(A):
    n = A.shape[0]
    kb = n // 512

    def body(a_ref, o_ref):
        k = pl.program_id(1)

        @pl.when(k == 0)
        def _():
            o_ref[...] = jnp.zeros_like(o_ref)

        o_ref[...] += jnp.sum(a_ref[...], axis=1)

        @pl.when(k == kb - 1)
        def _():
            o_ref[...] = lax.rsqrt(o_ref[...] + 1.0)

    return pl.pallas_call(
        body,
        grid=(n // 512, kb),
        in_specs=[pl.BlockSpec((512, 512), lambda i, k: (i, k))],
        out_specs=pl.BlockSpec((512,), lambda i, k: (i,)),
        out_shape=jax.ShapeDtypeStruct((n,), jnp.float32),
    )(A)


def _tc_gcn0_pre(x, W, incnt, selfcnt):
    n = x.shape[0]

    def body(x_ref, w_ref, ic_ref, sc_ref, g_ref, hs_ref, d_ref, f_ref):
        fill = jnp.where(sc_ref[...] == 0.0, 1.0, 0.0)
        dinv = lax.rsqrt(ic_ref[...] + fill)
        g = jnp.dot(x_ref[...], w_ref[...],
                    preferred_element_type=jnp.float32)
        g_ref[...] = g
        hs_ref[...] = g * dinv[:, None]
        d_ref[...] = dinv
        f_ref[...] = fill

    return pl.pallas_call(
        body,
        grid=(n // 512,),
        in_specs=[
            pl.BlockSpec((512, D), lambda i: (i, 0)),
            pl.BlockSpec((D, D), lambda i: (0, 0)),
            pl.BlockSpec((512,), lambda i: (i,)),
            pl.BlockSpec((512,), lambda i: (i,)),
        ],
        out_specs=[
            pl.BlockSpec((512, D), lambda i: (i, 0)),
            pl.BlockSpec((512, D), lambda i: (i, 0)),
            pl.BlockSpec((512,), lambda i: (i,)),
            pl.BlockSpec((512,), lambda i: (i,)),
        ],
        out_shape=[
            jax.ShapeDtypeStruct((n, D), jnp.float32),
            jax.ShapeDtypeStruct((n, D), jnp.float32),
            jax.ShapeDtypeStruct((n,), jnp.float32),
            jax.ShapeDtypeStruct((n,), jnp.float32),
        ],
    )(x, W, incnt, selfcnt)


def _tc_gcn0_post(z2, g, dinv, fill, b, W_next, relu, score_w=None):
    n = g.shape[0]

    def body(z_ref, g_ref, d_ref, f_ref, b_ref, w_ref, sw_ref, o1, o2):
        dinv = d_ref[...]
        zsum = z_ref[0] + z_ref[1]
        xx = dinv[:, None] * zsum \
            + (f_ref[...] * dinv * dinv)[:, None] * g_ref[...] + b_ref[...]
        if relu:
            xx = jnp.maximum(xx, 0.0)
        if score_w is None:
            gn = jnp.dot(xx, w_ref[...], preferred_element_type=jnp.float32)
            o1[...] = gn
            o2[...] = gn * dinv[:, None]
        else:
            o1[...] = xx
            w = sw_ref[...]
            nrm = jnp.sqrt(jnp.sum(w * w))
            o2[...] = jnp.tanh(
                jnp.dot(xx, (w / nrm).reshape(D, 1),
                        preferred_element_type=jnp.float32))

    w_next = W_next if W_next is not None else jnp.zeros((D, D), jnp.float32)
    sw = score_w if score_w is not None else jnp.zeros((1, D), jnp.float32)
    shapes = ([jax.ShapeDtypeStruct((n, D), jnp.float32),
               jax.ShapeDtypeStruct((n, D), jnp.float32)]
              if score_w is None else
              [jax.ShapeDtypeStruct((n, D), jnp.float32),
               jax.ShapeDtypeStruct((n, 1), jnp.float32)])
    o2_spec = (pl.BlockSpec((512, D), lambda i: (i, 0)) if score_w is None
               else pl.BlockSpec((512, 1), lambda i: (i, 0)))
    return pl.pallas_call(
        body,
        grid=(n // 512,),
        in_specs=[
            pl.BlockSpec((2, 512, D), lambda i: (0, i, 0)),
            pl.BlockSpec((512, D), lambda i: (i, 0)),
            pl.BlockSpec((512,), lambda i: (i,)),
            pl.BlockSpec((512,), lambda i: (i,)),
            pl.BlockSpec((1, D), lambda i: (0, 0)),
            pl.BlockSpec((D, D), lambda i: (0, 0)),
            pl.BlockSpec((1, D), lambda i: (0, 0)),
        ],
        out_specs=[
            pl.BlockSpec((512, D), lambda i: (i, 0)),
            o2_spec,
        ],
        out_shape=shapes,
    )(z2, g, dinv, fill, b.reshape(1, D), w_next, sw)


def _tc_final(x7, lin_W, lin_b):
    n = x7.shape[0]

    def body(x_ref, w_ref, b_ref, ge_ref, p_ref):
        ig = lax.broadcasted_iota(jnp.int32, (n, 1), 0)
        xm = jnp.where(ig < N2, x_ref[...], 0.0)
        ge = jnp.sum(xm, axis=0, keepdims=True) / jnp.float32(N2)
        ge_ref[...] = ge
        p_ref[...] = jnp.dot(ge, w_ref[...],
                             preferred_element_type=jnp.float32) + b_ref[...]

    return pl.pallas_call(
        body,
        in_specs=[
            pl.BlockSpec((n, D), lambda: (0, 0)),
            pl.BlockSpec((D, 64), lambda: (0, 0)),
            pl.BlockSpec((1, 64), lambda: (0, 0)),
        ],
        out_specs=[
            pl.BlockSpec((1, D), lambda: (0, 0)),
            pl.BlockSpec((1, 64), lambda: (0, 0)),
        ],
        out_shape=[
            jax.ShapeDtypeStruct((1, D), jnp.float32),
            jax.ShapeDtypeStruct((1, 64), jnp.float32),
        ],
    )(x7, lin_W, lin_b.reshape(1, 64))


def kernel(x, edge_index, conv_W, conv_b, pool_w, emb_W, emb_b, lin_W, lin_b):
    xpad = jnp.pad(x, ((0, NP0 - N0), (0, 0)))
    src = jnp.pad(edge_index[0], (0, EP - E))
    dst = jnp.pad(edge_index[1], (0, EP - E), constant_values=NP0)
    ones2 = jnp.stack([jnp.ones((128, 16), jnp.float32),
                       jnp.zeros((128, 16), jnp.float32)])
    zrow = jnp.zeros((128, D), jnp.float32)
    z16 = jnp.zeros((128, 16), jnp.float32)

    stats = _edge_stats(src, dst, ones2)
    incnt = stats[0, 0, :, 0] + stats[1, 0, :, 0]
    selfcnt = stats[0, 1, :, 0] + stats[1, 1, :, 0]

    g1, hs1, dinv0, fill0 = _tc_gcn0_pre(xpad, conv_W[0], incnt, selfcnt)
    z1 = _scatter_feat(src, dst, zrow, hs1)
    g2, hs2 = _tc_gcn0_post(z1, g1, dinv0, fill0, conv_b[0], conv_W[1],
                            relu=False)
    z2 = _scatter_feat(src, dst, zrow, hs2)
    x2, y1 = _tc_gcn0_post(z2, g2, dinv0, fill0, conv_b[1], None,
                           relu=True, score_w=pool_w[0].reshape(1, D))
    y1 = y1[:, 0]

    rank0, rank1m = _tc_rank(y1, N0, N1)
    perm1, vals1 = _tc_perm_vals(rank0, y1, NP1)
    Lf, Rf = _build_lr(src, dst, rank1m, perm1, z16)
    A1, A1T = _tc_mm_lr(Lf.reshape(NP1, NP0), Rf.reshape(NP0, NP1))
    dinv1 = _tc_deginv(A1)
    xp1 = _make_gather(NP1, D, False, 80)(x2, perm1)

    g3, gs3 = _tc_linear(xp1, conv_W[2], dinv1, vals=vals1)
    x3, _ = _tc_gcn_apply(A1, gs3, dinv1, conv_b[2], relu=False)
    g4, gs4 = _tc_linear(x3, conv_W[3], dinv1)
    x4, y2 = _tc_gcn_apply(A1, gs4, dinv1, conv_b[3], relu=True,
                           score_w=pool_w[1].reshape(1, D))

    rank2, _ = _tc_rank(y2, N1, N2)
    perm2, vals2 = _tc_perm_vals(rank2, y2, NP2)
    xp2 = _make_gather(NP2, D, False, 80)(x4, perm2)
    L2 = _make_gather(NP2, NP1, True, 16)(A1, perm2)
    R2T = _make_gather(NP2, NP1, True, 16)(A1T, perm2)
    A2 = _tc_mm_nt(L2, R2T)
    dinv2 = _tc_deginv(A2)

    g5, gs5 = _tc_linear(xp2, conv_W[4], dinv2, vals=vals2)
    x5, _ = _tc_gcn_apply(A2, gs5, dinv2, conv_b[4], relu=False)
    g6, gs6 = _tc_linear(x5, conv_W[5], dinv2)
    x6, _ = _tc_gcn_apply(A2, gs6, dinv2, conv_b[5], relu=True)

    g7, gs7 = _tc_linear(x6, emb_W, dinv2)
    x7, _ = _tc_gcn_apply(A2, gs7, dinv2, emb_b, relu=True)

    ge, preds = _tc_final(x7, lin_W, lin_b)
    return (preds, x7[:N2], ge)


# speedup vs baseline: 2.4697x; 2.4697x over previous
import functools

import jax
import jax.numpy as jnp
from jax import lax
from jax.experimental import pallas as pl
from jax.experimental.pallas import tpu as pltpu
from jax.experimental.pallas import tpu_sc as plsc

N0 = 10000
E = 160000
D = 128
NP0 = 10240
EP = 163840
N1, NP1 = 5000, 5120
N2, NP2 = 2500, 2560
NEG = -3.0e38

def _mesh():
    return plsc.VectorSubcoreMesh(core_axis_name="c", subcore_axis_name="s")


def _sc_edge_stats(src_h, dst_h, ones_h, out_h, sv, dv, mv, cnt_sh, self_sh,
                   ones_v):
    c = lax.axis_index("c")
    tid = lax.axis_index("s")
    wid = tid * 2 + c
    DUMP = NP0

    pltpu.sync_copy(ones_h.at[1], ones_v)
    for k in range(6):
        base = tid * 768 + k * 128
        pltpu.sync_copy(ones_v, cnt_sh.at[pl.ds(base, 128)])
        pltpu.sync_copy(ones_v, self_sh.at[pl.ds(base, 128)])
    pltpu.sync_copy(ones_h.at[0], ones_v)
    plsc.subcore_barrier()

    ebase = wid * 5120

    def body(b, carry):
        off = ebase + b * 128
        pltpu.sync_copy(src_h.at[pl.ds(off, 128)], sv)
        pltpu.sync_copy(dst_h.at[pl.ds(off, 128)], dv)
        pltpu.sync_copy(ones_v, cnt_sh.at[dv], add=True)
        for kk in range(8):
            s16 = sv[pl.ds(kk * 16, 16)]
            d16 = dv[pl.ds(kk * 16, 16)]
            sel = jnp.where(s16 == d16, d16, jnp.full((16,), DUMP, jnp.int32))
            mv[pl.ds(kk * 16, 16)] = sel
        pltpu.sync_copy(ones_v, self_sh.at[mv], add=True)
        return carry

    lax.fori_loop(0, 40, body, 0)
    plsc.subcore_barrier()
    lo = tid * 640
    pltpu.sync_copy(cnt_sh.at[pl.ds(lo, 640)], out_h.at[c, 0, pl.ds(lo, 640)])
    pltpu.sync_copy(self_sh.at[pl.ds(lo, 640)], out_h.at[c, 1, pl.ds(lo, 640)])


def _edge_stats(src, dst, ones2):
    k = functools.partial(
        pl.kernel, mesh=_mesh(),
        compiler_params=pltpu.CompilerParams(
            needs_layout_passes=False, use_tc_tiling_on_sc=False),
        out_type=jax.ShapeDtypeStruct((2, 2, NP0, 16), jnp.float32),
        scratch_types=[
            pltpu.VMEM((128,), jnp.int32),
            pltpu.VMEM((128,), jnp.int32),
            pltpu.VMEM((128,), jnp.int32),
            pltpu.VMEM_SHARED((12288, 16), jnp.float32),
            pltpu.VMEM_SHARED((12288, 16), jnp.float32),
            pltpu.VMEM((128, 16), jnp.float32),
        ],
    )(_sc_edge_stats)
    return k(src, dst, ones2)


def _sc_scatter_feat(src_h, dst_h, zeros_h, hs_h, out_h, sv, dv, rows_v,
                     zrow_v, zacc_sh, sem):
    c = lax.axis_index("c")
    tid = lax.axis_index("s")
    wid = tid * 2 + c

    pltpu.sync_copy(zeros_h, zrow_v)
    for k in range(5):
        base = tid * 648 + k * 128
        pltpu.sync_copy(zrow_v, zacc_sh.at[pl.ds(base, 128)])
    pltpu.sync_copy(zrow_v.at[pl.ds(0, 8)],
                    zacc_sh.at[pl.ds(tid * 648 + 640, 8)])
    plsc.subcore_barrier()

    ebase = wid * 5120

    def body(b, carry):
        off = ebase + b * 128
        pltpu.sync_copy(src_h.at[pl.ds(off, 128)], sv)
        pltpu.sync_copy(dst_h.at[pl.ds(off, 128)], dv)
        pltpu.async_copy(hs_h.at[sv], rows_v, sem).wait()
        pltpu.sync_copy(rows_v, zacc_sh.at[dv], add=True)
        return carry

    lax.fori_loop(0, 40, body, 0)
    plsc.subcore_barrier()
    lo = tid * 640
    pltpu.sync_copy(zacc_sh.at[pl.ds(lo, 640)], out_h.at[c, pl.ds(lo, 640)])


def _scatter_feat(src, dst, zeros_row, hs):
    k = functools.partial(
        pl.kernel, mesh=_mesh(),
        compiler_params=pltpu.CompilerParams(
            needs_layout_passes=False, use_tc_tiling_on_sc=False),
        out_type=jax.ShapeDtypeStruct((2, NP0, D), jnp.float32),
        scratch_types=[
            pltpu.VMEM((128,), jnp.int32),
            pltpu.VMEM((128,), jnp.int32),
            pltpu.VMEM((128, D), jnp.float32),
            pltpu.VMEM((128, D), jnp.float32),
            pltpu.VMEM_SHARED((10368, D), jnp.float32),
            pltpu.SemaphoreType.DMA,
        ],
    )(_sc_scatter_feat)
    return k(src, dst, zeros_row, hs)


_LROWS = 81920
_LBUF = 83968
_LDUMP = 81920
_NSTRIPES = 40


_KTOT = EP + NP1
_KPW = _KTOT // 32
_KPT = _KTOT // 16


def _sc_edge_keys(src_h, dst_h, rank_h, perm_h, kl_h, kr_h,
                  sv, dv, rank_v, perm_v, klb, krb):
    c = lax.axis_index("c")
    tid = lax.axis_index("s")
    wid = tid * 2 + c
    iota = lax.iota(jnp.int32, 16)
    neg16 = jnp.full((16,), -1, jnp.int32)

    pltpu.sync_copy(rank_h, rank_v)
    pltpu.sync_copy(perm_h, perm_v)
    ebase = wid * 5120

    def batch(bb, carry):
        pltpu.sync_copy(src_h.at[pl.ds(ebase + bb * 1024, 1024)], sv)
        pltpu.sync_copy(dst_h.at[pl.ds(ebase + bb * 1024, 1024)], dv)

        def ch_loop(ch, carry2):
            s16 = sv[pl.ds(ch * 16, 16)]
            d16 = dv[pl.ds(ch * 16, 16)]
            dc = jnp.minimum(d16, jnp.full((16,), NP0 - 1, jnp.int32))
            a16 = plsc.load_gather(rank_v, [dc])
            r16 = plsc.load_gather(rank_v, [s16])
            ok = s16 != d16
            klb[pl.ds(ch * 16, 16)] = jnp.where(
                ok & (a16 >= 0), a16 * NP0 + s16, neg16)
            krb[pl.ds(ch * 16, 16)] = jnp.where(
                ok & (r16 >= 0), d16 * NP1 + r16, neg16)
            return carry2

        lax.fori_loop(0, 64, ch_loop, 0)
        pltpu.sync_copy(klb, kl_h.at[pl.ds(ebase + bb * 1024, 1024)])
        pltpu.sync_copy(krb, kr_h.at[pl.ds(ebase + bb * 1024, 1024)])
        return carry

    lax.fori_loop(0, 5, batch, 0)

    def dch(ch, carry):
        aj = wid * 160 + ch * 16 + iota
        p16 = plsc.load_gather(perm_v, [aj])
        ok = aj < N1
        klb[pl.ds(ch * 16, 16)] = jnp.where(ok, aj * NP0 + p16, neg16)
        krb[pl.ds(ch * 16, 16)] = jnp.where(ok, p16 * NP1 + aj, neg16)
        return carry

    lax.fori_loop(0, 10, dch, 0)
    dbase = EP + wid * 160
    pltpu.sync_copy(klb.at[pl.ds(0, 160)], kl_h.at[pl.ds(dbase, 160)])
    pltpu.sync_copy(krb.at[pl.ds(0, 160)], kr_h.at[pl.ds(dbase, 160)])


def _edge_keys(src, dst, rank1, perm1):
    k = functools.partial(
        pl.kernel, mesh=_mesh(),
        compiler_params=pltpu.CompilerParams(
            needs_layout_passes=False, use_tc_tiling_on_sc=False),
        out_type=[
            jax.ShapeDtypeStruct((_KTOT,), jnp.int32),
            jax.ShapeDtypeStruct((_KTOT,), jnp.int32),
        ],
        scratch_types=[
            pltpu.VMEM((1024,), jnp.int32),
            pltpu.VMEM((1024,), jnp.int32),
            pltpu.VMEM((NP0,), jnp.int32),
            pltpu.VMEM((NP1,), jnp.int32),
            pltpu.VMEM((1024,), jnp.int32),
            pltpu.VMEM((1024,), jnp.int32),
        ],
    )(_sc_edge_keys)
    return k(src, dst, rank1, perm1)


def _sc_build_lr(kl_h, kr_h, zeros16_h, lout_h, rout_h,
                 kb_v, cidx_v, dbuf_v, z16_v, lr_sh):
    c = lax.axis_index("c")
    tid = lax.axis_index("s")
    ones16 = jnp.full((16,), 1.0, jnp.float32)
    zero16 = jnp.zeros((16,), jnp.float32)
    dump16 = jnp.full((16,), _LDUMP, jnp.int32)

    pltpu.sync_copy(zeros16_h, z16_v)
    kbase = tid * _KPT

    def clean(hi_row):
        del hi_row
        for i in range(8):
            for j in range(16):
                dbuf_v[i * 16 + j] = zero16

    def reset_cidx():
        for k in range(8):
            cidx_v[pl.ds(k * 16, 16)] = dump16

    reset_cidx()
    clean(128)

    def stripe(st, carry):
        for k in range(41):
            base = tid * 5248 + k * 128
            pltpu.sync_copy(z16_v, lr_sh.at[pl.ds(base, 128)])
        plsc.subcore_barrier()
        lo = st * (_LROWS * 16)
        hi = lo + _LROWS * 16

        def kbatch(bb, cur0):
            off = kbase + bb * 1056

            @pl.when(c == 0)
            def _():
                pltpu.sync_copy(kl_h.at[pl.ds(off, 1056)], kb_v)

            @pl.when(c == 1)
            def _():
                pltpu.sync_copy(kr_h.at[pl.ds(off, 1056)], kb_v)

            def chunk(ch, cur):
                @pl.when(cur >= 112)
                def _():
                    pltpu.sync_copy(dbuf_v, lr_sh.at[cidx_v], add=True)
                    reset_cidx()
                    clean(128)

                cur = jnp.where(cur >= 112, jnp.int32(0), cur)
                k16 = kb_v[pl.ds(ch * 16, 16)]
                m = (k16 >= lo) & (k16 < hi)
                rel = k16 - lo
                fl = jnp.right_shift(rel, 4)
                ln = jnp.bitwise_and(rel, 15)
                cs = plsc.cumsum(jnp.where(m, jnp.full((16,), 1, jnp.int32),
                                           jnp.zeros((16,), jnp.int32)))
                nv = jnp.max(cs)
                slot = cur + cs - 1
                plsc.store_scatter(cidx_v, [slot], fl, mask=m)
                plsc.store_scatter(dbuf_v, [slot, ln], ones16, mask=m)
                return cur + nv

            return lax.fori_loop(0, 66, chunk, cur0)

        cur = lax.fori_loop(0, 10, kbatch, jnp.int32(0))
        plsc.subcore_barrier()
        pltpu.sync_copy(dbuf_v, lr_sh.at[cidx_v], add=True)
        reset_cidx()
        clean(128)
        plsc.subcore_barrier()
        tlo = tid * 5120

        @pl.when(c == 0)
        def _():
            pltpu.sync_copy(lr_sh.at[pl.ds(tlo, 5120)],
                            lout_h.at[pl.ds(st * _LROWS + tlo, 5120)])

        @pl.when(c == 1)
        def _():
            pltpu.sync_copy(lr_sh.at[pl.ds(tlo, 5120)],
                            rout_h.at[pl.ds(st * _LROWS + tlo, 5120)])

        plsc.subcore_barrier()
        return carry

    lax.fori_loop(0, _NSTRIPES, stripe, 0)


def _build_lr(kl, kr, zeros16):
    k = functools.partial(
        pl.kernel, mesh=_mesh(),
        compiler_params=pltpu.CompilerParams(
            needs_layout_passes=False, use_tc_tiling_on_sc=False),
        out_type=[
            jax.ShapeDtypeStruct((NP1 * NP0 // 16, 16), jnp.float32),
            jax.ShapeDtypeStruct((NP0 * NP1 // 16, 16), jnp.float32),
        ],
        scratch_types=[
            pltpu.VMEM((1056,), jnp.int32),
            pltpu.VMEM((128,), jnp.int32),
            pltpu.VMEM((128, 16), jnp.float32),
            pltpu.VMEM((128, 16), jnp.float32),
            pltpu.VMEM_SHARED((83968, 16), jnp.float32),
        ],
    )(_sc_build_lr)
    return k(kl, kr, zeros16)


def _make_gather(n_rows, n_cols, diag, batch):
    per_w = n_rows // 32
    nb = per_w // batch

    def body(table_h, idx_h, out_h, idx_v, rows_v, sem):
        iota = lax.iota(jnp.int32, 16)
        c = lax.axis_index("c")
        tid = lax.axis_index("s")
        wid = tid * 2 + c
        base = wid * per_w
        pltpu.sync_copy(idx_h.at[pl.ds(base, per_w)], idx_v)

        def b_loop(b, carry):
            pltpu.async_copy(table_h.at[idx_v.at[pl.ds(b * batch, batch)]],
                             rows_v, sem).wait()
            if diag:
                for kk in range(batch // 16):
                    rows16 = kk * 16 + iota
                    cols16 = idx_v[pl.ds(b * batch + kk * 16, 16)]
                    plsc.addupdate_scatter(
                        rows_v, [rows16, cols16],
                        jnp.full((16,), 1.0, jnp.float32))
            pltpu.sync_copy(rows_v, out_h.at[pl.ds(base + b * batch, batch)])
            return carry

        lax.fori_loop(0, nb, b_loop, 0)

    k = functools.partial(
        pl.kernel, mesh=_mesh(),
        compiler_params=pltpu.CompilerParams(
            needs_layout_passes=False, use_tc_tiling_on_sc=False),
        out_type=jax.ShapeDtypeStruct((n_rows, n_cols), jnp.float32),
        scratch_types=[
            pltpu.VMEM((per_w,), jnp.int32),
            pltpu.VMEM((batch, n_cols), jnp.float32),
            pltpu.SemaphoreType.DMA,
        ],
    )(body)
    return k


def _tc_linear(xin, W, dinv, vals=None):
    n = xin.shape[0]

    def body(x_ref, w_ref, d_ref, v_ref, g_ref, gs_ref):
        xb = x_ref[...]
        if vals is not None:
            xb = xb * v_ref[...][:, None]
        g = jnp.dot(xb, w_ref[...], preferred_element_type=jnp.float32)
        g_ref[...] = g
        gs_ref[...] = g * d_ref[...][:, None]

    vv = vals if vals is not None else dinv
    return pl.pallas_call(
        body,
        grid=(n // 512,),
        in_specs=[
            pl.BlockSpec((512, D), lambda i: (i, 0)),
            pl.BlockSpec((D, D), lambda i: (0, 0)),
            pl.BlockSpec((512,), lambda i: (i,)),
            pl.BlockSpec((512,), lambda i: (i,)),
        ],
        out_specs=[
            pl.BlockSpec((512, D), lambda i: (i, 0)),
            pl.BlockSpec((512, D), lambda i: (i, 0)),
        ],
        out_shape=[
            jax.ShapeDtypeStruct((n, D), jnp.float32),
            jax.ShapeDtypeStruct((n, D), jnp.float32),
        ],
    )(xin, W, dinv, vv)


def _tc_gcn_apply(A, gs, dinv, b, relu, score_w=None):
    n = A.shape[0]
    kb = n // 512

    def body(a_ref, gs_ref, gsb_ref, d_ref, b_ref, w_ref, o_ref, y_ref):
        k = pl.program_id(1)

        @pl.when(k == 0)
        def _():
            o_ref[...] = jnp.zeros_like(o_ref)

        o_ref[...] += jnp.dot(a_ref[...], gs_ref[...],
                              preferred_element_type=jnp.float32)

        @pl.when(k == kb - 1)
        def _():
            out = d_ref[...][:, None] * (o_ref[...] + gsb_ref[...]) \
                + b_ref[...]
            if relu:
                out = jnp.maximum(out, 0.0)
            o_ref[...] = out
            if score_w is not None:
                w = w_ref[...]
                nrm = jnp.sqrt(jnp.sum(w * w))
                y_ref[...] = jnp.tanh(
                    jnp.dot(out, (w / nrm).reshape(D, 1),
                            preferred_element_type=jnp.float32))

    w_in = score_w if score_w is not None else jnp.zeros((1, D), jnp.float32)
    out, y = pl.pallas_call(
        body,
        grid=(n // 512, kb),
        in_specs=[
            pl.BlockSpec((512, 512), lambda i, k: (i, k)),
            pl.BlockSpec((512, D), lambda i, k: (k, 0)),
            pl.BlockSpec((512, D), lambda i, k: (i, 0)),
            pl.BlockSpec((512,), lambda i, k: (i,)),
            pl.BlockSpec((1, D), lambda i, k: (0, 0)),
            pl.BlockSpec((1, D), lambda i, k: (0, 0)),
        ],
        out_specs=[
            pl.BlockSpec((512, D), lambda i, k: (i, 0)),
            pl.BlockSpec((512, 1), lambda i, k: (i, 0)),
        ],
        out_shape=[
            jax.ShapeDtypeStruct((n, D), jnp.float32),
            jax.ShapeDtypeStruct((n, 1), jnp.float32),
        ],
    )(A, gs, gs, dinv, b.reshape(1, D), w_in)
    return out, y[:, 0]


def _tc_rank(y, n_valid, k_keep):
    n = y.shape[0]
    ch = 1024
    nch = n // ch

    def body(yi_ref, yf_ref, r_ref, rm_ref):
        i = pl.program_id(0)
        ig = i * 512 + lax.broadcasted_iota(jnp.int32, (512,), 0)
        yiv = jnp.where(ig < n_valid, yi_ref[...], NEG)

        def cbody(cc, acc):
            yj = yf_ref[pl.ds(cc * ch, ch)]
            jg = cc * ch + lax.broadcasted_iota(jnp.int32, (ch,), 0)
            yjv = jnp.where(jg < n_valid, yj, NEG)
            gt = yjv[None, :] > yiv[:, None]
            tie = (yjv[None, :] == yiv[:, None]) & (jg[None, :] < ig[:, None])
            return acc + jnp.sum((gt | tie).astype(jnp.int32), axis=1)

        r = lax.fori_loop(0, nch, cbody, jnp.zeros((512,), jnp.int32))
        r = jnp.where(ig < n_valid, r, jnp.full((512,), n, jnp.int32))
        r_ref[...] = r
        rm_ref[...] = jnp.where(r < k_keep, r,
                                jnp.full((512,), -1, jnp.int32))

    return pl.pallas_call(
        body,
        grid=(n // 512,),
        in_specs=[
            pl.BlockSpec((512,), lambda i: (i,)),
            pl.BlockSpec((n,), lambda i: (0,)),
        ],
        out_specs=[
            pl.BlockSpec((512,), lambda i: (i,)),
            pl.BlockSpec((512,), lambda i: (i,)),
        ],
        out_shape=[
            jax.ShapeDtypeStruct((n,), jnp.int32),
            jax.ShapeDtypeStruct((n,), jnp.int32),
        ],
    )(y, y)


def _tc_perm_vals(rank, y, np_out):
    n = rank.shape[0]
    ch = 1024
    nch = n // ch

    def body(r_ref, y_ref, p_ref, v_ref):
        j = pl.program_id(0)
        jg = j * 512 + lax.broadcasted_iota(jnp.int32, (512,), 0)

        def cbody(cc, acc):
            pacc, vacc = acc
            rr = r_ref[pl.ds(cc * ch, ch)]
            yy = y_ref[pl.ds(cc * ch, ch)]
            ig = cc * ch + lax.broadcasted_iota(jnp.int32, (ch,), 0)
            eq = (rr[:, None] == jg[None, :]).astype(jnp.float32)
            pacc = pacc + jnp.sum(eq * ig[:, None].astype(jnp.float32),
                                  axis=0)
            vacc = vacc + jnp.sum(eq * yy[:, None], axis=0)
            return (pacc, vacc)

        p, v = lax.fori_loop(0, nch, cbody,
                             (jnp.zeros((512,), jnp.float32),
                              jnp.zeros((512,), jnp.float32)))
        p_ref[...] = p.astype(jnp.int32)
        v_ref[...] = v

    return pl.pallas_call(
        body,
        grid=(np_out // 512,),
        in_specs=[
            pl.BlockSpec((n,), lambda j: (0,)),
            pl.BlockSpec((n,), lambda j: (0,)),
        ],
        out_specs=[
            pl.BlockSpec((512,), lambda j: (j,)),
            pl.BlockSpec((512,), lambda j: (j,)),
        ],
        out_shape=[
            jax.ShapeDtypeStruct((np_out,), jnp.int32),
            jax.ShapeDtypeStruct((np_out,), jnp.float32),
        ],
    )(rank, y)


def _tc_mm_lr(Lf, Rf):
    M, K = NP1, NP0
    BM, BN, BK = 512, 512, 1024
    gk = K // BK

    def body(l_ref, r_ref, o_ref, t_ref):
        k = pl.program_id(2)

        @pl.when(k == 0)
        def _():
            o_ref[...] = jnp.zeros_like(o_ref)

        o_ref[...] += jnp.dot(l_ref[...].astype(jnp.bfloat16),
                              r_ref[...].astype(jnp.bfloat16),
                              preferred_element_type=jnp.float32)

        @pl.when(k == gk - 1)
        def _():
            i = pl.program_id(0)
            j = pl.program_id(1)
            rg = i * BM + lax.broadcasted_iota(jnp.int32, (BM, BN), 0)
            cg = j * BN + lax.broadcasted_iota(jnp.int32, (BM, BN), 1)
            acc = jnp.where(rg == cg, 0.0, o_ref[...])
            o_ref[...] = acc
            t_ref[...] = acc.T

    return pl.pallas_call(
        body,
        grid=(M // BM, M // BN, gk),
        in_specs=[
            pl.BlockSpec((BM, BK), lambda i, j, k: (i, k)),
            pl.BlockSpec((BK, BN), lambda i, j, k: (k, j)),
        ],
        out_specs=[
            pl.BlockSpec((BM, BN), lambda i, j, k: (i, j)),
            pl.BlockSpec((BN, BM), lambda i, j, k: (j, i)),
        ],
        out_shape=[
            jax.ShapeDtypeStruct((M, M), jnp.float32),
            jax.ShapeDtypeStruct((M, M), jnp.float32),
        ],
    )(Lf, Rf)


def _tc_mm_nt(L2, R2T):
    M, K = NP2, NP1
    BM, BN, BK = 512, 512, 1024
    gk = K // BK

    def body(l_ref, r_ref, o_ref):
        k = pl.program_id(2)

        @pl.when(k == 0)
        def _():
            o_ref[...] = jnp.zeros_like(o_ref)

        o_ref[...] += lax.dot_general(
            l_ref[...].astype(jnp.bfloat16), r_ref[...].astype(jnp.bfloat16),
            (((1,), (1,)), ((), ())), preferred_element_type=jnp.float32)

        @pl.when(k == gk - 1)
        def _():
            i = pl.program_id(0)
            j = pl.program_id(1)
            rg = i * BM + lax.broadcasted_iota(jnp.int32, (BM, BN), 0)
            cg = j * BN + lax.broadcasted_iota(jnp.int32, (BM, BN), 1)
            bad = (rg == cg) | (rg >= N2) | (cg >= N2)
            o_ref[...] = jnp.where(bad, 0.0, o_ref[...])

    return pl.pallas_call(
        body,
        grid=(M // BM, M // BN, gk),
        in_specs=[
            pl.BlockSpec((BM, BK), lambda i, j, k: (i, k)),
            pl.BlockSpec((BN, BK), lambda i, j, k: (j, k)),
        ],
        out_specs=pl.BlockSpec((BM, BN), lambda i, j, k: (i, j)),
        out_shape=jax.ShapeDtypeStruct((M, M), jnp.float32),
    )(L2, R2T)


def _tc_deginv(A):
    n = A.shape[0]
    kb = n // 512

    def body(a_ref, o_ref):
        k = pl.program_id(1)

        @pl.when(k == 0)
        def _():
            o_ref[...] = jnp.zeros_like(o_ref)

        o_ref[...] += jnp.sum(a_ref[...], axis=1)

        @pl.when(k == kb - 1)
        def _():
            o_ref[...] = lax.rsqrt(o_ref[...] + 1.0)

    return pl.pallas_call(
        body,
        grid=(n // 512, kb),
        in_specs=[pl.BlockSpec((512, 512), lambda i, k: (i, k))],
        out_specs=pl.BlockSpec((512,), lambda i, k: (i,)),
        out_shape=jax.ShapeDtypeStruct((n,), jnp.float32),
    )(A)


def _tc_gcn0_pre(x, W, incnt, selfcnt):
    n = x.shape[0]

    def body(x_ref, w_ref, ic_ref, sc_ref, g_ref, hs_ref, d_ref, f_ref):
        fill = jnp.where(sc_ref[...] == 0.0, 1.0, 0.0)
        dinv = lax.rsqrt(ic_ref[...] + fill)
        g = jnp.dot(x_ref[...], w_ref[...],
                    preferred_element_type=jnp.float32)
        g_ref[...] = g
        hs_ref[...] = g * dinv[:, None]
        d_ref[...] = dinv
        f_ref[...] = fill

    return pl.pallas_call(
        body,
        grid=(n // 512,),
        in_specs=[
            pl.BlockSpec((512, D), lambda i: (i, 0)),
            pl.BlockSpec((D, D), lambda i: (0, 0)),
            pl.BlockSpec((512,), lambda i: (i,)),
            pl.BlockSpec((512,), lambda i: (i,)),
        ],
        out_specs=[
            pl.BlockSpec((512, D), lambda i: (i, 0)),
            pl.BlockSpec((512, D), lambda i: (i, 0)),
            pl.BlockSpec((512,), lambda i: (i,)),
            pl.BlockSpec((512,), lambda i: (i,)),
        ],
        out_shape=[
            jax.ShapeDtypeStruct((n, D), jnp.float32),
            jax.ShapeDtypeStruct((n, D), jnp.float32),
            jax.ShapeDtypeStruct((n,), jnp.float32),
            jax.ShapeDtypeStruct((n,), jnp.float32),
        ],
    )(x, W, incnt, selfcnt)


def _tc_gcn0_post(z2, g, dinv, fill, b, W_next, relu, score_w=None):
    n = g.shape[0]

    def body(z_ref, g_ref, d_ref, f_ref, b_ref, w_ref, sw_ref, o1, o2):
        dinv = d_ref[...]
        zsum = z_ref[0] + z_ref[1]
        xx = dinv[:, None] * zsum \
            + (f_ref[...] * dinv * dinv)[:, None] * g_ref[...] + b_ref[...]
        if relu:
            xx = jnp.maximum(xx, 0.0)
        if score_w is None:
            gn = jnp.dot(xx, w_ref[...], preferred_element_type=jnp.float32)
            o1[...] = gn
            o2[...] = gn * dinv[:, None]
        else:
            o1[...] = xx
            w = sw_ref[...]
            nrm = jnp.sqrt(jnp.sum(w * w))
            o2[...] = jnp.tanh(
                jnp.dot(xx, (w / nrm).reshape(D, 1),
                        preferred_element_type=jnp.float32))

    w_next = W_next if W_next is not None else jnp.zeros((D, D), jnp.float32)
    sw = score_w if score_w is not None else jnp.zeros((1, D), jnp.float32)
    shapes = ([jax.ShapeDtypeStruct((n, D), jnp.float32),
               jax.ShapeDtypeStruct((n, D), jnp.float32)]
              if score_w is None else
              [jax.ShapeDtypeStruct((n, D), jnp.float32),
               jax.ShapeDtypeStruct((n, 1), jnp.float32)])
    o2_spec = (pl.BlockSpec((512, D), lambda i: (i, 0)) if score_w is None
               else pl.BlockSpec((512, 1), lambda i: (i, 0)))
    return pl.pallas_call(
        body,
        grid=(n // 512,),
        in_specs=[
            pl.BlockSpec((2, 512, D), lambda i: (0, i, 0)),
            pl.BlockSpec((512, D), lambda i: (i, 0)),
            pl.BlockSpec((512,), lambda i: (i,)),
            pl.BlockSpec((512,), lambda i: (i,)),
            pl.BlockSpec((1, D), lambda i: (0, 0)),
            pl.BlockSpec((D, D), lambda i: (0, 0)),
            pl.BlockSpec((1, D), lambda i: (0, 0)),
        ],
        out_specs=[
            pl.BlockSpec((512, D), lambda i: (i, 0)),
            o2_spec,
        ],
        out_shape=shapes,
    )(z2, g, dinv, fill, b.reshape(1, D), w_next, sw)


def _tc_final(x7, lin_W, lin_b):
    n = x7.shape[0]

    def body(x_ref, w_ref, b_ref, ge_ref, p_ref):
        ig = lax.broadcasted_iota(jnp.int32, (n, 1), 0)
        xm = jnp.where(ig < N2, x_ref[...], 0.0)
        ge = jnp.sum(xm, axis=0, keepdims=True) / jnp.float32(N2)
        ge_ref[...] = ge
        p_ref[...] = jnp.dot(ge, w_ref[...],
                             preferred_element_type=jnp.float32) + b_ref[...]

    return pl.pallas_call(
        body,
        in_specs=[
            pl.BlockSpec((n, D), lambda: (0, 0)),
            pl.BlockSpec((D, 64), lambda: (0, 0)),
            pl.BlockSpec((1, 64), lambda: (0, 0)),
        ],
        out_specs=[
            pl.BlockSpec((1, D), lambda: (0, 0)),
            pl.BlockSpec((1, 64), lambda: (0, 0)),
        ],
        out_shape=[
            jax.ShapeDtypeStruct((1, D), jnp.float32),
            jax.ShapeDtypeStruct((1, 64), jnp.float32),
        ],
    )(x7, lin_W, lin_b.reshape(1, 64))


def kernel(x, edge_index, conv_W, conv_b, pool_w, emb_W, emb_b, lin_W, lin_b):
    xpad = jnp.pad(x, ((0, NP0 - N0), (0, 0)))
    src = jnp.pad(edge_index[0], (0, EP - E))
    dst = jnp.pad(edge_index[1], (0, EP - E), constant_values=NP0)
    ones2 = jnp.stack([jnp.ones((128, 16), jnp.float32),
                       jnp.zeros((128, 16), jnp.float32)])
    zrow = jnp.zeros((128, D), jnp.float32)
    z16 = jnp.zeros((128, 16), jnp.float32)

    _STAGE = 0
    stats = _edge_stats(src, dst, ones2)
    incnt = stats[0, 0, :, 0] + stats[1, 0, :, 0]
    selfcnt = stats[0, 1, :, 0] + stats[1, 1, :, 0]

    g1, hs1, dinv0, fill0 = _tc_gcn0_pre(xpad, conv_W[0], incnt, selfcnt)
    z1 = _scatter_feat(src, dst, zrow, hs1)
    if _STAGE == 2:
        s = jnp.sum(z1) * 0.0
        return (jnp.zeros((1, 64)) + s, jnp.zeros((N2, D)) + s,
                jnp.zeros((1, D)) + s)
    g2, hs2 = _tc_gcn0_post(z1, g1, dinv0, fill0, conv_b[0], conv_W[1],
                            relu=False)
    z2 = _scatter_feat(src, dst, zrow, hs2)
    x2, y1 = _tc_gcn0_post(z2, g2, dinv0, fill0, conv_b[1], None,
                           relu=True, score_w=pool_w[0].reshape(1, D))
    y1 = y1[:, 0]

    rank0, rank1m = _tc_rank(y1, N0, N1)
    perm1, vals1 = _tc_perm_vals(rank0, y1, NP1)
    kl, kr = _edge_keys(src, dst, rank1m, perm1)
    if _STAGE == 35:
        s = (jnp.sum(kl) + jnp.sum(kr)).astype(jnp.float32) * 0.0
        return (jnp.zeros((1, 64)) + s, jnp.zeros((N2, D)) + s,
                jnp.zeros((1, D)) + s)
    Lf, Rf = _build_lr(kl, kr, z16)
    if _STAGE == 3:
        s = (jnp.sum(Lf) + jnp.sum(Rf)) * 0.0
        return (jnp.zeros((1, 64)) + s, jnp.zeros((N2, D)) + s,
                jnp.zeros((1, D)) + s)
    A1, A1T = _tc_mm_lr(Lf.reshape(NP1, NP0), Rf.reshape(NP0, NP1))
    dinv1 = _tc_deginv(A1)
    xp1 = _make_gather(NP1, D, False, 80)(x2, perm1)

    g3, gs3 = _tc_linear(xp1, conv_W[2], dinv1, vals=vals1)
    x3, _ = _tc_gcn_apply(A1, gs3, dinv1, conv_b[2], relu=False)
    g4, gs4 = _tc_linear(x3, conv_W[3], dinv1)
    x4, y2 = _tc_gcn_apply(A1, gs4, dinv1, conv_b[3], relu=True,
                           score_w=pool_w[1].reshape(1, D))

    rank2, _ = _tc_rank(y2, N1, N2)
    perm2, vals2 = _tc_perm_vals(rank2, y2, NP2)
    xp2 = _make_gather(NP2, D, False, 80)(x4, perm2)
    L2 = _make_gather(NP2, NP1, True, 16)(A1, perm2)
    R2T = _make_gather(NP2, NP1, True, 16)(A1T, perm2)
    A2 = _tc_mm_nt(L2, R2T)
    dinv2 = _tc_deginv(A2)

    g5, gs5 = _tc_linear(xp2, conv_W[4], dinv2, vals=vals2)
    x5, _ = _tc_gcn_apply(A2, gs5, dinv2, conv_b[4], relu=False)
    g6, gs6 = _tc_linear(x5, conv_W[5], dinv2)
    x6, _ = _tc_gcn_apply(A2, gs6, dinv2, conv_b[5], relu=True)

    g7, gs7 = _tc_linear(x6, emb_W, dinv2)
    x7, _ = _tc_gcn_apply(A2, gs7, dinv2, emb_b, relu=True)

    ge, preds = _tc_final(x7, lin_W, lin_b)
    return (preds, x7[:N2], ge)
